# Initial kernel scaffold; baseline (speedup 1.0000x reference)
import jax
import jax.numpy as jnp
from jax.experimental import pallas as pl


def kernel(x, pilots):
    raise NotImplementedError("write your pallas kernel here")


# speedup vs baseline: 9.8440x; 9.8440x over previous
import functools

import jax
import jax.numpy as jnp
from jax import lax
from jax.experimental import pallas as pl
from jax.experimental.pallas import tpu as pltpu
from jax.experimental.pallas import tpu_sc as plsc

BATCH = 64
NUM_TX = 4
NUM_STREAMS = 2
NUM_OFDM = 14
FFT = 4096
NUM_DATA = 12
PAIRS = BATCH * NUM_TX * NUM_STREAMS
NUM_WORKERS = 32
PAIRS_PER_W = PAIRS // NUM_WORKERS
IN_ROW = NUM_DATA * FFT
OUT_ROW = NUM_OFDM * FFT


def _sc_grid_map(x1, p1):
    mesh = plsc.VectorSubcoreMesh(core_axis_name="c", subcore_axis_name="s")

    @functools.partial(
        pl.kernel,
        mesh=mesh,
        out_type=jax.ShapeDtypeStruct((PAIRS * OUT_ROW,), jnp.float32),
        scratch_types=[
            pltpu.VMEM((OUT_ROW,), jnp.float32),
            pltpu.VMEM((OUT_ROW,), jnp.float32),
            pltpu.SemaphoreType.DMA,
            pltpu.SemaphoreType.DMA,
            pltpu.SemaphoreType.DMA,
            pltpu.SemaphoreType.DMA,
        ],
    )
    def grid_map(x_hbm, p_hbm, out_hbm, buf0, buf1, in0, in1, out0, out1):
        wid = lax.axis_index("s") * 2 + lax.axis_index("c")
        base = wid * PAIRS_PER_W
        bufs = (buf0, buf1)
        in_sems = (in0, in1)
        out_sems = (out0, out1)
        out_waits = [None, None]
        for j in range(PAIRS_PER_W):
            slot = j % 2
            buf = bufs[slot]
            sem = in_sems[slot]
            xb = pl.multiple_of((base + j) * IN_ROW, FFT)
            ob = pl.multiple_of((base + j) * OUT_ROW, FFT)
            q = j % (NUM_TX * NUM_STREAMS)
            if out_waits[slot] is not None:
                out_waits[slot].wait()
            copies = (
                pltpu.async_copy(x_hbm.at[pl.ds(xb, 2 * FFT)], buf.at[pl.ds(0, 2 * FFT)], sem),
                pltpu.async_copy(x_hbm.at[pl.ds(xb + 2 * FFT, 8 * FFT)], buf.at[pl.ds(3 * FFT, 8 * FFT)], sem),
                pltpu.async_copy(x_hbm.at[pl.ds(xb + 10 * FFT, 2 * FFT)], buf.at[pl.ds(12 * FFT, 2 * FFT)], sem),
                pltpu.async_copy(p_hbm.at[pl.ds(q * 2 * FFT, FFT)], buf.at[pl.ds(2 * FFT, FFT)], sem),
                pltpu.async_copy(p_hbm.at[pl.ds(q * 2 * FFT + FFT, FFT)], buf.at[pl.ds(11 * FFT, FFT)], sem),
            )
            for c in copies:
                c.wait()
            out_waits[slot] = pltpu.async_copy(buf, out_hbm.at[pl.ds(ob, OUT_ROW)], out_sems[slot])
        for w in out_waits:
            w.wait()

    return grid_map(x1, p1)


def kernel(x, pilots):
    rg = _sc_grid_map(x.reshape(-1), pilots.reshape(-1))
    return rg.reshape(BATCH, NUM_TX, NUM_STREAMS, NUM_OFDM, FFT)
